# baseline (speedup 1.0000x reference)
import functools

import jax
import jax.numpy as jnp
from jax import lax
from jax.experimental import pallas as pl
from jax.experimental.pallas import tpu as pltpu
from jax.experimental.pallas import tpu_sc as plsc

B = 16384
H = 64
V = 100000

_info = plsc.get_sparse_core_info()
NC = _info.num_cores
NS = _info.num_subcores
NW = NC * NS
BPW = B // NW
CH = 128
NCHUNK = BPW // CH
NBUF = 2

_mesh = plsc.VectorSubcoreMesh(core_axis_name="c", subcore_axis_name="s")


_CC_R = 16384


def _cc_body(at, bt, out):
    out[...] = jnp.concatenate(
        [jnp.transpose(at[...]), jnp.transpose(bt[...])], axis=1)


def _tc_concat(at, bt):
    return pl.pallas_call(
        _cc_body,
        grid=(pl.cdiv(V, _CC_R),),
        in_specs=[
            pl.BlockSpec((H, _CC_R), lambda i: (0, i)),
            pl.BlockSpec((H, _CC_R), lambda i: (0, i)),
        ],
        out_specs=pl.BlockSpec((_CC_R, 2 * H), lambda i: (i, 0)),
        out_shape=jax.ShapeDtypeStruct((V, 2 * H), jnp.float32),
    )(at, bt)


@functools.partial(
    pl.kernel,
    mesh=_mesh,
    out_type=jax.ShapeDtypeStruct((B, 2 * H), jnp.float32),
    scratch_types=[
        pltpu.VMEM((BPW,), jnp.int32),
        pltpu.VMEM((NBUF, CH, 2 * H), jnp.float32),
        pltpu.SemaphoreType.DMA,
        pltpu.SemaphoreType.DMA,
    ],
)
def _sc_gather(ids, cat, out, idv, buf, sem0, sem1):
    wid = lax.axis_index("s") * NC + lax.axis_index("c")
    base = wid * BPW
    pltpu.sync_copy(ids.at[pl.ds(base, BPW)], idv)
    sems = (sem0, sem1)

    def issue(k):
        return pltpu.async_copy(cat.at[idv.at[pl.ds(k * CH, CH)]],
                                buf.at[k % NBUF], sems[k % NBUF])

    pending = {k: issue(k) for k in range(min(NBUF, NCHUNK))}
    for k in range(NCHUNK):
        pending.pop(k).wait()
        off = base + k * CH
        pltpu.sync_copy(buf.at[k % NBUF], out.at[pl.ds(off, CH)])
        if k + NBUF < NCHUNK:
            pending[k + NBUF] = issue(k + NBUF)


_TC_BLK = 4096


def _tc_body(outc, oute, w1e, w1c, b1, wmf, wmlp, cb, out):
    mfp = outc[:, :H] * oute[:, :H]
    mc = outc[:, H:]
    me = oute[:, H:]
    h = jnp.dot(me, w1e[...], preferred_element_type=jnp.float32)
    h = h + jnp.dot(mc, w1c[...], preferred_element_type=jnp.float32)
    h = jnp.maximum(h + b1[...], 0.0)
    z = (jnp.dot(mfp, wmf[...], preferred_element_type=jnp.float32)
         + jnp.dot(h, wmlp[...], preferred_element_type=jnp.float32)
         + cb[0, 0])
    out[...] = jax.nn.sigmoid(z)


def _tc_head(outc, oute, w1e, w1c, b1, wmf, wmlp, cb):
    grid = (B // _TC_BLK,)
    return pl.pallas_call(
        _tc_body,
        grid=grid,
        in_specs=[
            pl.BlockSpec((_TC_BLK, 2 * H), lambda i: (i, 0)),
            pl.BlockSpec((_TC_BLK, 2 * H), lambda i: (i, 0)),
            pl.BlockSpec((H, H), lambda i: (0, 0)),
            pl.BlockSpec((H, H), lambda i: (0, 0)),
            pl.BlockSpec((1, H), lambda i: (0, 0)),
            pl.BlockSpec((H, 1), lambda i: (0, 0)),
            pl.BlockSpec((H, 1), lambda i: (0, 0)),
            pl.BlockSpec((1, 1), lambda i: (0, 0)),
        ],
        out_specs=pl.BlockSpec((_TC_BLK, 1), lambda i: (i, 0)),
        out_shape=jax.ShapeDtypeStruct((B, 1), jnp.float32),
    )(outc, oute, w1e, w1c, b1, wmf, wmlp, cb)


def kernel(compound_ids, enzyme_ids, mf_c, mf_e, mlp_c, mlp_e,
           fc1_w, fc1_b, ce_w, ce_b):
    cids = compound_ids.astype(jnp.int32)
    eids = enzyme_ids.astype(jnp.int32)
    cat_c = _tc_concat(mf_c.T, mlp_c.T)
    outc = _sc_gather(cids, cat_c)
    cat_e = _tc_concat(mf_e.T, mlp_e.T)
    oute = _sc_gather(eids, cat_e)
    w1e = fc1_w[:, :H].T
    w1c = fc1_w[:, H:].T
    b1 = fc1_b.reshape(1, H)
    wmf = ce_w[:, :H].T
    wmlp = ce_w[:, H:].T
    cb = ce_b.reshape(1, 1)
    return _tc_head(outc, oute, w1e, w1c, b1, wmf, wmlp, cb)


# speedup vs baseline: 1.0028x; 1.0028x over previous
import functools

import jax
import jax.numpy as jnp
from jax import lax
from jax.experimental import pallas as pl
from jax.experimental.pallas import tpu as pltpu
from jax.experimental.pallas import tpu_sc as plsc

B = 16384
H = 64
V = 100000

_info = plsc.get_sparse_core_info()
NC = _info.num_cores
NS = _info.num_subcores
NW = NC * NS
BPW = B // NW
CH = 128
NCHUNK = BPW // CH
NBUF = 2

_mesh = plsc.VectorSubcoreMesh(core_axis_name="c", subcore_axis_name="s")


_CC_R = 8192


def _cc_body(at, bt, out):
    out[...] = jnp.concatenate(
        [jnp.transpose(at[...]), jnp.transpose(bt[...])], axis=1)


def _tc_concat(at, bt):
    return pl.pallas_call(
        _cc_body,
        grid=(pl.cdiv(V, _CC_R),),
        in_specs=[
            pl.BlockSpec((H, _CC_R), lambda i: (0, i)),
            pl.BlockSpec((H, _CC_R), lambda i: (0, i)),
        ],
        out_specs=pl.BlockSpec((_CC_R, 2 * H), lambda i: (i, 0)),
        out_shape=jax.ShapeDtypeStruct((V, 2 * H), jnp.float32),
    )(at, bt)


@functools.partial(
    pl.kernel,
    mesh=_mesh,
    out_type=jax.ShapeDtypeStruct((B, 2 * H), jnp.float32),
    scratch_types=[
        pltpu.VMEM((BPW,), jnp.int32),
        pltpu.VMEM((NBUF, CH, 2 * H), jnp.float32),
        pltpu.SemaphoreType.DMA,
        pltpu.SemaphoreType.DMA,
    ],
)
def _sc_gather(ids, cat, out, idv, buf, sem0, sem1):
    wid = lax.axis_index("s") * NC + lax.axis_index("c")
    base = wid * BPW
    pltpu.sync_copy(ids.at[pl.ds(base, BPW)], idv)
    sems = (sem0, sem1)

    def issue(k):
        return pltpu.async_copy(cat.at[idv.at[pl.ds(k * CH, CH)]],
                                buf.at[k % NBUF], sems[k % NBUF])

    pending = {k: issue(k) for k in range(min(NBUF, NCHUNK))}
    for k in range(NCHUNK):
        pending.pop(k).wait()
        off = base + k * CH
        pltpu.sync_copy(buf.at[k % NBUF], out.at[pl.ds(off, CH)])
        if k + NBUF < NCHUNK:
            pending[k + NBUF] = issue(k + NBUF)


_TC_BLK = 4096


def _tc_body(outc, oute, w1e, w1c, b1, wmf, wmlp, cb, out):
    mfp = outc[:, :H] * oute[:, :H]
    mc = outc[:, H:]
    me = oute[:, H:]
    h = jnp.dot(me, w1e[...], preferred_element_type=jnp.float32)
    h = h + jnp.dot(mc, w1c[...], preferred_element_type=jnp.float32)
    h = jnp.maximum(h + b1[...], 0.0)
    z = (jnp.dot(mfp, wmf[...], preferred_element_type=jnp.float32)
         + jnp.dot(h, wmlp[...], preferred_element_type=jnp.float32)
         + cb[0, 0])
    out[...] = jax.nn.sigmoid(z)


def _tc_head(outc, oute, w1e, w1c, b1, wmf, wmlp, cb):
    grid = (B // _TC_BLK,)
    return pl.pallas_call(
        _tc_body,
        grid=grid,
        in_specs=[
            pl.BlockSpec((_TC_BLK, 2 * H), lambda i: (i, 0)),
            pl.BlockSpec((_TC_BLK, 2 * H), lambda i: (i, 0)),
            pl.BlockSpec((H, H), lambda i: (0, 0)),
            pl.BlockSpec((H, H), lambda i: (0, 0)),
            pl.BlockSpec((1, H), lambda i: (0, 0)),
            pl.BlockSpec((H, 1), lambda i: (0, 0)),
            pl.BlockSpec((H, 1), lambda i: (0, 0)),
            pl.BlockSpec((1, 1), lambda i: (0, 0)),
        ],
        out_specs=pl.BlockSpec((_TC_BLK, 1), lambda i: (i, 0)),
        out_shape=jax.ShapeDtypeStruct((B, 1), jnp.float32),
    )(outc, oute, w1e, w1c, b1, wmf, wmlp, cb)


def kernel(compound_ids, enzyme_ids, mf_c, mf_e, mlp_c, mlp_e,
           fc1_w, fc1_b, ce_w, ce_b):
    cids = compound_ids.astype(jnp.int32)
    eids = enzyme_ids.astype(jnp.int32)
    cat_c = _tc_concat(mf_c.T, mlp_c.T)
    outc = _sc_gather(cids, cat_c)
    cat_e = _tc_concat(mf_e.T, mlp_e.T)
    oute = _sc_gather(eids, cat_e)
    w1e = fc1_w[:, :H].T
    w1c = fc1_w[:, H:].T
    b1 = fc1_b.reshape(1, H)
    wmf = ce_w[:, :H].T
    wmlp = ce_w[:, H:].T
    cb = ce_b.reshape(1, 1)
    return _tc_head(outc, oute, w1e, w1c, b1, wmf, wmlp, cb)


# speedup vs baseline: 1.2322x; 1.2288x over previous
import functools

import jax
import jax.numpy as jnp
from jax import lax
from jax.experimental import pallas as pl
from jax.experimental.pallas import tpu as pltpu
from jax.experimental.pallas import tpu_sc as plsc

B = 16384
H = 64
V = 100000

_info = plsc.get_sparse_core_info()
NC = _info.num_cores
NS = _info.num_subcores
NW = NC * NS
BPW = B // NW
CH = 128
NCHUNK = BPW // CH
NBUF = 2

_mesh = plsc.VectorSubcoreMesh(core_axis_name="c", subcore_axis_name="s")


_CC_R = 8192


def _bf16_bits(x):
    return lax.bitcast_convert_type(
        x.astype(jnp.bfloat16), jnp.uint16).astype(jnp.uint32)


def _cc_body(at, bt, ct, dt, out):
    lo = jnp.concatenate(
        [jnp.transpose(at[...]), jnp.transpose(bt[...])], axis=1)
    hi = jnp.concatenate(
        [jnp.transpose(ct[...]), jnp.transpose(dt[...])], axis=1)
    out[...] = lax.bitcast_convert_type(
        _bf16_bits(lo) | (_bf16_bits(hi) << 16), jnp.int32)


def _tc_concat(at, bt, ct, dt):
    ispec = pl.BlockSpec((H, _CC_R), lambda i: (0, i))
    return pl.pallas_call(
        _cc_body,
        grid=(pl.cdiv(V, _CC_R),),
        in_specs=[ispec, ispec, ispec, ispec],
        out_specs=pl.BlockSpec((_CC_R, 2 * H), lambda i: (i, 0)),
        out_shape=jax.ShapeDtypeStruct((V, 2 * H), jnp.int32),
    )(at, bt, ct, dt)


@functools.partial(
    pl.kernel,
    mesh=_mesh,
    out_type=jax.ShapeDtypeStruct((B, 2 * H), jnp.int32),
    scratch_types=[
        pltpu.VMEM((BPW,), jnp.int32),
        pltpu.VMEM((NBUF, CH, 2 * H), jnp.int32),
        pltpu.SemaphoreType.DMA,
        pltpu.SemaphoreType.DMA,
    ],
)
def _sc_gather(ids, cat, out, idv, buf, sem0, sem1):
    wid = lax.axis_index("s") * NC + lax.axis_index("c")
    base = wid * BPW
    pltpu.sync_copy(ids.at[pl.ds(base, BPW)], idv)
    sems = (sem0, sem1)

    def issue(k):
        return pltpu.async_copy(cat.at[idv.at[pl.ds(k * CH, CH)]],
                                buf.at[k % NBUF], sems[k % NBUF])

    pending = {k: issue(k) for k in range(min(NBUF, NCHUNK))}
    for k in range(NCHUNK):
        pending.pop(k).wait()
        off = base + k * CH
        pltpu.sync_copy(buf.at[k % NBUF], out.at[pl.ds(off, CH)])
        if k + NBUF < NCHUNK:
            pending[k + NBUF] = issue(k + NBUF)


_TC_BLK = 4096


def _tc_body(outc, oute, w1e, w1c, b1, wmf, wmlp, cb, out):
    wc = lax.bitcast_convert_type(outc[...], jnp.uint32)
    we = lax.bitcast_convert_type(oute[...], jnp.uint32)
    c_rows = lax.bitcast_convert_type(
        (wc & jnp.uint32(0xFFFF)).astype(jnp.uint16), jnp.bfloat16
    ).astype(jnp.float32)
    e_rows = lax.bitcast_convert_type(
        (we >> jnp.uint32(16)).astype(jnp.uint16), jnp.bfloat16
    ).astype(jnp.float32)
    mfp = c_rows[:, :H] * e_rows[:, :H]
    mc = c_rows[:, H:]
    me = e_rows[:, H:]
    h = jnp.dot(me, w1e[...], preferred_element_type=jnp.float32)
    h = h + jnp.dot(mc, w1c[...], preferred_element_type=jnp.float32)
    h = jnp.maximum(h + b1[...], 0.0)
    z = (jnp.dot(mfp, wmf[...], preferred_element_type=jnp.float32)
         + jnp.dot(h, wmlp[...], preferred_element_type=jnp.float32)
         + cb[0, 0])
    out[...] = jax.nn.sigmoid(z)


def _tc_head(outc, oute, w1e, w1c, b1, wmf, wmlp, cb):
    grid = (B // _TC_BLK,)
    return pl.pallas_call(
        _tc_body,
        grid=grid,
        in_specs=[
            pl.BlockSpec((_TC_BLK, 2 * H), lambda i: (i, 0)),
            pl.BlockSpec((_TC_BLK, 2 * H), lambda i: (i, 0)),
            pl.BlockSpec((H, H), lambda i: (0, 0)),
            pl.BlockSpec((H, H), lambda i: (0, 0)),
            pl.BlockSpec((1, H), lambda i: (0, 0)),
            pl.BlockSpec((H, 1), lambda i: (0, 0)),
            pl.BlockSpec((H, 1), lambda i: (0, 0)),
            pl.BlockSpec((1, 1), lambda i: (0, 0)),
        ],
        out_specs=pl.BlockSpec((_TC_BLK, 1), lambda i: (i, 0)),
        out_shape=jax.ShapeDtypeStruct((B, 1), jnp.float32),
    )(outc, oute, w1e, w1c, b1, wmf, wmlp, cb)


def kernel(compound_ids, enzyme_ids, mf_c, mf_e, mlp_c, mlp_e,
           fc1_w, fc1_b, ce_w, ce_b):
    cids = compound_ids.astype(jnp.int32)
    eids = enzyme_ids.astype(jnp.int32)
    cat = _tc_concat(mf_c.T, mlp_c.T, mf_e.T, mlp_e.T)
    outc = _sc_gather(cids, cat)
    oute = _sc_gather(eids, cat)
    w1e = fc1_w[:, :H].T
    w1c = fc1_w[:, H:].T
    b1 = fc1_b.reshape(1, H)
    wmf = ce_w[:, :H].T
    wmlp = ce_w[:, H:].T
    cb = ce_b.reshape(1, 1)
    return _tc_head(outc, oute, w1e, w1c, b1, wmf, wmlp, cb)


# speedup vs baseline: 1.2944x; 1.0505x over previous
import functools

import jax
import jax.numpy as jnp
from jax import lax
from jax.experimental import pallas as pl
from jax.experimental.pallas import tpu as pltpu
from jax.experimental.pallas import tpu_sc as plsc

B = 16384
H = 64
V = 100000

_info = plsc.get_sparse_core_info()
NC = _info.num_cores
NS = _info.num_subcores
NW = NC * NS
BPW = B // NW
CH = 128
NCHUNK = BPW // CH
NBUF = 2

_mesh = plsc.VectorSubcoreMesh(core_axis_name="c", subcore_axis_name="s")


_CC_R = 8192


def _bf16_bits(x):
    return lax.bitcast_convert_type(
        x.astype(jnp.bfloat16), jnp.uint16).astype(jnp.uint32)


def _cc_body(at, bt, ct, dt, out):
    lo = jnp.concatenate(
        [jnp.transpose(at[...]), jnp.transpose(bt[...])], axis=1)
    hi = jnp.concatenate(
        [jnp.transpose(ct[...]), jnp.transpose(dt[...])], axis=1)
    out[...] = lax.bitcast_convert_type(
        _bf16_bits(lo) | (_bf16_bits(hi) << 16), jnp.int32)


def _tc_concat(at, bt, ct, dt):
    ispec = pl.BlockSpec((H, _CC_R), lambda i: (0, i))
    return pl.pallas_call(
        _cc_body,
        grid=(pl.cdiv(V, _CC_R),),
        in_specs=[ispec, ispec, ispec, ispec],
        out_specs=pl.BlockSpec((_CC_R, 2 * H), lambda i: (i, 0)),
        out_shape=jax.ShapeDtypeStruct((V, 2 * H), jnp.int32),
    )(at, bt, ct, dt)


@functools.partial(
    pl.kernel,
    mesh=_mesh,
    out_type=[
        jax.ShapeDtypeStruct((B, 2 * H), jnp.int32),
        jax.ShapeDtypeStruct((B, 2 * H), jnp.int32),
    ],
    scratch_types=[
        pltpu.VMEM((BPW,), jnp.int32),
        pltpu.VMEM((BPW,), jnp.int32),
        pltpu.VMEM((NBUF, CH, 2 * H), jnp.int32),
        pltpu.VMEM((NBUF, CH, 2 * H), jnp.int32),
        pltpu.SemaphoreType.DMA,
        pltpu.SemaphoreType.DMA,
    ],
)
def _sc_gather(cids, eids, cat, outc, oute, idc, ide, bufc, bufe, sem0, sem1):
    wid = lax.axis_index("s") * NC + lax.axis_index("c")
    base = wid * BPW
    pltpu.sync_copy(cids.at[pl.ds(base, BPW)], idc)
    pltpu.sync_copy(eids.at[pl.ds(base, BPW)], ide)
    sems = (sem0, sem1)

    def issue(k):
        s = sems[k % NBUF]
        islc = pl.ds(k * CH, CH)
        return (
            pltpu.async_copy(cat.at[idc.at[islc]], bufc.at[k % NBUF], s),
            pltpu.async_copy(cat.at[ide.at[islc]], bufe.at[k % NBUF], s),
        )

    pending = {k: issue(k) for k in range(min(NBUF, NCHUNK))}
    for k in range(NCHUNK):
        ca, ce = pending.pop(k)
        ca.wait()
        ce.wait()
        off = base + k * CH
        pltpu.sync_copy(bufc.at[k % NBUF], outc.at[pl.ds(off, CH)])
        pltpu.sync_copy(bufe.at[k % NBUF], oute.at[pl.ds(off, CH)])
        if k + NBUF < NCHUNK:
            pending[k + NBUF] = issue(k + NBUF)


_TC_BLK = 4096


def _tc_body(outc, oute, w1e, w1c, b1, wmf, wmlp, cb, out):
    wc = lax.bitcast_convert_type(outc[...], jnp.uint32)
    we = lax.bitcast_convert_type(oute[...], jnp.uint32)
    c_rows = lax.bitcast_convert_type(
        (wc & jnp.uint32(0xFFFF)).astype(jnp.uint16), jnp.bfloat16
    ).astype(jnp.float32)
    e_rows = lax.bitcast_convert_type(
        (we >> jnp.uint32(16)).astype(jnp.uint16), jnp.bfloat16
    ).astype(jnp.float32)
    mfp = c_rows[:, :H] * e_rows[:, :H]
    mc = c_rows[:, H:]
    me = e_rows[:, H:]
    h = jnp.dot(me, w1e[...], preferred_element_type=jnp.float32)
    h = h + jnp.dot(mc, w1c[...], preferred_element_type=jnp.float32)
    h = jnp.maximum(h + b1[...], 0.0)
    z = (jnp.dot(mfp, wmf[...], preferred_element_type=jnp.float32)
         + jnp.dot(h, wmlp[...], preferred_element_type=jnp.float32)
         + cb[0, 0])
    out[...] = jax.nn.sigmoid(z)


def _tc_head(outc, oute, w1e, w1c, b1, wmf, wmlp, cb):
    grid = (B // _TC_BLK,)
    return pl.pallas_call(
        _tc_body,
        grid=grid,
        in_specs=[
            pl.BlockSpec((_TC_BLK, 2 * H), lambda i: (i, 0)),
            pl.BlockSpec((_TC_BLK, 2 * H), lambda i: (i, 0)),
            pl.BlockSpec((H, H), lambda i: (0, 0)),
            pl.BlockSpec((H, H), lambda i: (0, 0)),
            pl.BlockSpec((1, H), lambda i: (0, 0)),
            pl.BlockSpec((H, 1), lambda i: (0, 0)),
            pl.BlockSpec((H, 1), lambda i: (0, 0)),
            pl.BlockSpec((1, 1), lambda i: (0, 0)),
        ],
        out_specs=pl.BlockSpec((_TC_BLK, 1), lambda i: (i, 0)),
        out_shape=jax.ShapeDtypeStruct((B, 1), jnp.float32),
    )(outc, oute, w1e, w1c, b1, wmf, wmlp, cb)


def kernel(compound_ids, enzyme_ids, mf_c, mf_e, mlp_c, mlp_e,
           fc1_w, fc1_b, ce_w, ce_b):
    cids = compound_ids.astype(jnp.int32)
    eids = enzyme_ids.astype(jnp.int32)
    cat = _tc_concat(mf_c.T, mlp_c.T, mf_e.T, mlp_e.T)
    outc, oute = _sc_gather(cids, eids, cat)
    w1e = fc1_w[:, :H].T
    w1c = fc1_w[:, H:].T
    b1 = fc1_b.reshape(1, H)
    wmf = ce_w[:, :H].T
    wmlp = ce_w[:, H:].T
    cb = ce_b.reshape(1, 1)
    return _tc_head(outc, oute, w1e, w1c, b1, wmf, wmlp, cb)
